# baseline (speedup 1.0000x reference)
import jax
import jax.numpy as jnp
from jax.experimental import pallas as pl


def _gelu(x):
    return 0.5 * x * (1.0 + jnp.tanh(0.7978845608028654 * (x + 0.044715 * x * x * x)))


def _one_batch(h, mrowf, w1, w3, W2, b2, tri_f, trib, iis, jjs):
    s = h.shape[0]
    mrowb = mrowf > 0.0

    cdims = (((1,), (1,)), ((), ()))
    sl = jax.lax.dot_general(w1, h, cdims, preferred_element_type=jnp.float32)
    a_row = jax.lax.dot_general(w3, h, cdims,
                                preferred_element_type=jnp.float32)
    sc = _gelu(jnp.dot(h, W2, preferred_element_type=jnp.float32) + b2)
    c_row = jax.lax.dot_general(w3, sc, cdims,
                                preferred_element_type=jnp.float32)

    slm = mrowf * sl + (mrowf - 1.0) * 10.0
    m1 = jnp.max(slm)
    z1 = jnp.sum(jnp.exp(slm - m1))

    neg = jnp.float32(-1e30)
    ma = jnp.max(jnp.where(mrowb, a_row, neg))
    mc = jnp.max(jnp.where(mrowb, c_row, neg))
    m2 = jnp.maximum(ma + mc, -10.0)
    ea = jnp.where(mrowb, jnp.exp(a_row - ma), 0.0)
    ec = jnp.where(mrowb, jnp.exp(c_row - mc), 0.0)

    pad = jnp.zeros_like(mrowf)
    stack = jnp.concatenate(
        [mrowf, slm, c_row, pad, pad, pad, pad, pad], axis=0)
    colsT = jnp.transpose(stack, (1, 0))
    mcolb = colsT[:, 0:1] > 0.0
    slm_c = colsT[:, 1:2]
    c_c = colsT[:, 2:3]

    sa_row = jax.lax.dot_general(ea, tri_f, (((1,), (1,)), ((), ())),
                                 preferred_element_type=jnp.float32)
    z2p = jnp.sum(ec * sa_row)
    p = jnp.sum(mrowf)
    npairs = 0.5 * p * (p + 1.0)
    z2 = z2p * jnp.exp((ma + mc) - m2) \
        + (s * s - npairs) * jnp.exp(-10.0 - m2)
    lse2 = m2 + jnp.log(z2)

    ut_c = (m1 + jnp.log(z1) + lse2 + 10.0) - slm_c
    d_c = c_c + 10.0

    vb = trib & (mcolb & mrowb)
    return ut_c - jnp.where(vb, d_c + a_row, 0.0)


def _answer_finder_kernel(x_ref, mr_ref, W0_ref, b0_ref, w1_ref,
                          W2_ref, b2_ref, w3_ref, out_ref):
    nb = x_ref.shape[0]
    s = out_ref.shape[1]
    w1 = w1_ref[...]
    w3 = w3_ref[...]
    W2 = W2_ref[...]
    b2 = b2_ref[...]

    xall = x_ref[...].reshape(nb * s, x_ref.shape[2])
    hall = _gelu(jnp.dot(xall.astype(jnp.bfloat16),
                         W0_ref[...].astype(jnp.bfloat16),
                         preferred_element_type=jnp.float32) + b0_ref[...])

    ii = jax.lax.broadcasted_iota(jnp.int32, (s, s), 0)
    jj = jax.lax.broadcasted_iota(jnp.int32, (s, s), 1)
    trib = jj >= ii
    tri_f = jnp.where(trib, 1.0, 0.0)

    for bb in range(nb):
        h = hall[bb * s:(bb + 1) * s, :]
        mrowf = mr_ref[bb].astype(jnp.float32)
        out_ref[bb] = _one_batch(h, mrowf, w1, w3, W2, b2,
                                 tri_f, trib, ii, jj)


@jax.jit
def kernel(inputs, mask, W0, b0, w1, W2, b2, w3):
    B, S, D = inputs.shape
    U = W0.shape[1]
    NB = 1
    mr = mask.reshape(B, 1, S)
    return pl.pallas_call(
        _answer_finder_kernel,
        grid=(B // NB,),
        in_specs=[
            pl.BlockSpec((NB, S, D), lambda b: (b, 0, 0)),
            pl.BlockSpec((NB, 1, S), lambda b: (b, 0, 0)),
            pl.BlockSpec((D, U), lambda b: (0, 0)),
            pl.BlockSpec((1, U), lambda b: (0, 0)),
            pl.BlockSpec((1, U), lambda b: (0, 0)),
            pl.BlockSpec((U, U), lambda b: (0, 0)),
            pl.BlockSpec((1, U), lambda b: (0, 0)),
            pl.BlockSpec((1, U), lambda b: (0, 0)),
        ],
        out_specs=pl.BlockSpec((NB, S, S), lambda b: (b, 0, 0)),
        out_shape=jax.ShapeDtypeStruct((B, S, S), jnp.float32),
    )(inputs, mr, W0, b0.reshape(1, U), w1.reshape(1, U),
      W2, b2.reshape(1, U), w3.reshape(1, U))


# speedup vs baseline: 1.0654x; 1.0654x over previous
import jax
import jax.numpy as jnp
from jax.experimental import pallas as pl


def _gelu(x):
    return 0.5 * x * (1.0 + jnp.tanh(0.7978845608028654 * (x + 0.044715 * x * x * x)))


def _one_batch(h, mrowf, w1, w3, W2, b2, tri_f, trib, iis, jjs):
    s = h.shape[0]
    mrowb = mrowf > 0.0

    cdims = (((1,), (1,)), ((), ()))
    w13 = jnp.concatenate([w1, w3], axis=0)
    sla = jax.lax.dot_general(w13, h, cdims,
                              preferred_element_type=jnp.float32)
    sl = sla[0:1, :]
    a_row = sla[1:2, :]
    sc = _gelu(jnp.dot(h, W2, preferred_element_type=jnp.float32) + b2)
    c_row = jax.lax.dot_general(w3, sc, cdims,
                                preferred_element_type=jnp.float32)

    slm = mrowf * sl + (mrowf - 1.0) * 10.0
    m1 = jnp.max(slm)
    z1 = jnp.sum(jnp.exp(slm - m1))

    neg = jnp.float32(-1e30)
    ma = jnp.max(jnp.where(mrowb, a_row, neg))
    mc = jnp.max(jnp.where(mrowb, c_row, neg))
    m2 = jnp.maximum(ma + mc, -10.0)
    ea = jnp.where(mrowb, jnp.exp(a_row - ma), 0.0)
    ec = jnp.where(mrowb, jnp.exp(c_row - mc), 0.0)

    pad = jnp.zeros_like(mrowf)
    stack = jnp.concatenate(
        [mrowf, slm, c_row, pad, pad, pad, pad, pad], axis=0)
    colsT = jnp.transpose(stack, (1, 0))
    mcolb = colsT[:, 0:1] > 0.0
    slm_c = colsT[:, 1:2]
    c_c = colsT[:, 2:3]

    sa_row = jax.lax.dot_general(ea, tri_f, (((1,), (1,)), ((), ())),
                                 preferred_element_type=jnp.float32)
    z2p = jnp.sum(ec * sa_row)
    p = jnp.sum(mrowf)
    npairs = 0.5 * p * (p + 1.0)
    z2 = z2p * jnp.exp((ma + mc) - m2) \
        + (s * s - npairs) * jnp.exp(-10.0 - m2)
    lse2 = m2 + jnp.log(z2)

    d_c = c_c + 10.0
    vb = trib & (mcolb & mrowb)
    tmp = slm_c + jnp.where(vb, d_c + a_row, 0.0)
    k = m1 + jnp.log(z1) + lse2 + 10.0
    return k - tmp


def _answer_finder_kernel(x_ref, mr_ref, W0_ref, b0_ref, w1_ref,
                          W2_ref, b2_ref, w3_ref, out_ref):
    nb = x_ref.shape[0]
    s = out_ref.shape[1]
    w1 = w1_ref[...]
    w3 = w3_ref[...]
    W2 = W2_ref[...]
    b2 = b2_ref[...]

    xall = x_ref[...].reshape(nb * s, x_ref.shape[2])
    hall = _gelu(jnp.dot(xall.astype(jnp.bfloat16),
                         W0_ref[...].astype(jnp.bfloat16),
                         preferred_element_type=jnp.float32) + b0_ref[...])

    ii = jax.lax.broadcasted_iota(jnp.int32, (s, s), 0)
    jj = jax.lax.broadcasted_iota(jnp.int32, (s, s), 1)
    trib = jj >= ii
    tri_f = jnp.where(trib, 1.0, 0.0)

    for bb in range(nb):
        h = hall[bb * s:(bb + 1) * s, :]
        mrowf = mr_ref[bb].astype(jnp.float32)
        out_ref[bb] = _one_batch(h, mrowf, w1, w3, W2, b2,
                                 tri_f, trib, ii, jj)


@jax.jit
def kernel(inputs, mask, W0, b0, w1, W2, b2, w3):
    B, S, D = inputs.shape
    U = W0.shape[1]
    NB = 2
    mr = mask.reshape(B, 1, S)
    return pl.pallas_call(
        _answer_finder_kernel,
        grid=(B // NB,),
        in_specs=[
            pl.BlockSpec((NB, S, D), lambda b: (b, 0, 0)),
            pl.BlockSpec((NB, 1, S), lambda b: (b, 0, 0)),
            pl.BlockSpec((D, U), lambda b: (0, 0)),
            pl.BlockSpec((1, U), lambda b: (0, 0)),
            pl.BlockSpec((1, U), lambda b: (0, 0)),
            pl.BlockSpec((U, U), lambda b: (0, 0)),
            pl.BlockSpec((1, U), lambda b: (0, 0)),
            pl.BlockSpec((1, U), lambda b: (0, 0)),
        ],
        out_specs=pl.BlockSpec((NB, S, S), lambda b: (b, 0, 0)),
        out_shape=jax.ShapeDtypeStruct((B, S, S), jnp.float32),
    )(inputs, mr, W0, b0.reshape(1, U), w1.reshape(1, U),
      W2, b2.reshape(1, U), w3.reshape(1, U))
